# Initial kernel scaffold; baseline (speedup 1.0000x reference)
import jax
import jax.numpy as jnp
from jax.experimental import pallas as pl


def kernel(x, edge_index, W1_self, W1_neigh, b1, W2_self, W2_neigh, b2):
    raise NotImplementedError("write your pallas kernel here")


# speedup vs baseline: 4.4966x; 4.4966x over previous
import jax
import jax.numpy as jnp
from jax import lax
from jax.experimental import pallas as pl
from jax.experimental.pallas import tpu as pltpu
from jax.experimental.pallas import tpu_sc as plsc

N = 10000
E = 320000
D = 128
NC = 2
NS = 16
NW = NC * NS
K = 80
EW = E // NW
NCHUNK = EW // K
RPT = N // NS

_MESH = plsc.VectorSubcoreMesh(
    core_axis_name="c", subcore_axis_name="s", num_cores=NC, num_subcores=NS
)


def _make_sc_agg():
  scratch_types = [
      pltpu.VMEM((K,), jnp.int32),
      pltpu.VMEM((K,), jnp.int32),
      pltpu.VMEM((K, D), jnp.float32),
      pltpu.SemaphoreType.DMA,
      pltpu.VMEM_SHARED((N, D), jnp.float32),
  ]

  def body(x, srcr, dstr, zrows, agg_out, idx_s, idx_d, rows, sem, agg_sh):
    c = lax.axis_index("c")
    s = lax.axis_index("s")
    wid = c * NS + s
    pltpu.sync_copy(zrows, agg_sh.at[pl.ds(s * RPT, RPT)])
    plsc.subcore_barrier()

    ebase = wid * EW

    def chunk(i, carry):
      base = ebase + i * K
      pltpu.sync_copy(srcr.at[pl.ds(base, K)], idx_s)
      pltpu.sync_copy(dstr.at[pl.ds(base, K)], idx_d)
      pltpu.async_copy(x.at[idx_s], rows, sem).wait()
      pltpu.sync_copy(rows, agg_sh.at[idx_d], add=True)
      return carry

    lax.fori_loop(0, NCHUNK, chunk, 0)
    plsc.subcore_barrier()
    pltpu.sync_copy(agg_sh.at[pl.ds(s * RPT, RPT)], agg_out.at[c, s])

  return pl.kernel(
      body, mesh=_MESH, scratch_types=scratch_types, name="sage_sc_agg",
      out_type=[jax.ShapeDtypeStruct((NC, NS, RPT, D), jnp.float32)],
  )


def _make_sc_deg():
  scratch_types = [
      pltpu.VMEM((K,), jnp.int32),
      pltpu.VMEM((K, D), jnp.float32),
      pltpu.VMEM_SHARED((N, D), jnp.float32),
  ]

  def body(dstr, zrows, onesr, deg_out, idx_d, ones_v, deg_sh):
    c = lax.axis_index("c")
    s = lax.axis_index("s")
    wid = c * NS + s
    pltpu.sync_copy(zrows, deg_sh.at[pl.ds(s * RPT, RPT)])
    pltpu.sync_copy(onesr, ones_v)
    plsc.subcore_barrier()

    ebase = wid * EW

    def chunk(i, carry):
      base = ebase + i * K
      pltpu.sync_copy(dstr.at[pl.ds(base, K)], idx_d)
      pltpu.sync_copy(ones_v, deg_sh.at[idx_d], add=True)
      return carry

    lax.fori_loop(0, NCHUNK, chunk, 0)
    plsc.subcore_barrier()
    pltpu.sync_copy(deg_sh.at[pl.ds(s * RPT, RPT)], deg_out.at[c, s])

  return pl.kernel(
      body, mesh=_MESH, scratch_types=scratch_types, name="sage_sc_deg",
      out_type=[jax.ShapeDtypeStruct((NC, NS, RPT, D), jnp.float32)],
  )


def _make_dense(relu: bool):
  R = 1000

  def body(x_ref, a0_ref, a1_ref, d0_ref, d1_ref, ws_ref, wn_ref, b_ref,
           o_ref):
    deg = jnp.maximum(d0_ref[:, 0:1] + d1_ref[:, 0:1], 1.0)
    agg = (a0_ref[...] + a1_ref[...]) / deg
    dn = (((1,), (1,)), ((), ()))
    h = (
        lax.dot_general(x_ref[...], ws_ref[...], dn,
                        preferred_element_type=jnp.float32)
        + lax.dot_general(agg, wn_ref[...], dn,
                          preferred_element_type=jnp.float32)
        + b_ref[...]
    )
    o_ref[...] = jnp.maximum(h, 0.0) if relu else h

  row_block = lambda i: (i, 0)
  fixed = lambda i: (0, 0)
  return pl.pallas_call(
      body,
      grid=(N // R,),
      in_specs=[
          pl.BlockSpec((R, D), row_block),
          pl.BlockSpec((R, D), row_block),
          pl.BlockSpec((R, D), row_block),
          pl.BlockSpec((R, D), row_block),
          pl.BlockSpec((R, D), row_block),
          pl.BlockSpec((D, D), fixed),
          pl.BlockSpec((D, D), fixed),
          pl.BlockSpec((1, D), fixed),
      ],
      out_specs=pl.BlockSpec((R, D), row_block),
      out_shape=jax.ShapeDtypeStruct((N, D), jnp.float32),
  )


def kernel(x, edge_index, W1_self, W1_neigh, b1, W2_self, W2_neigh, b2):
  src = edge_index[0].astype(jnp.int32)
  dst = edge_index[1].astype(jnp.int32)
  zrows = jnp.zeros((RPT, D), jnp.float32)
  ones_rows = jnp.ones((K, D), jnp.float32)

  sc_agg = _make_sc_agg()

  (deg,) = _make_sc_deg()(dst, zrows, ones_rows)
  deg = deg.reshape(NC, N, D)

  (agg1,) = sc_agg(x, src, dst, zrows)
  agg1 = agg1.reshape(NC, N, D)
  h = _make_dense(True)(x, agg1[0], agg1[1], deg[0], deg[1],
                        W1_self, W1_neigh, b1.reshape(1, D))

  (agg2,) = sc_agg(h, src, dst, zrows)
  agg2 = agg2.reshape(NC, N, D)
  out = _make_dense(False)(h, agg2[0], agg2[1], deg[0], deg[1],
                           W2_self, W2_neigh, b2.reshape(1, D))
  return out


# speedup vs baseline: 9.0954x; 2.0228x over previous
import jax
import jax.numpy as jnp
from jax import lax
from jax.experimental import pallas as pl
from jax.experimental.pallas import tpu as pltpu
from jax.experimental.pallas import tpu_sc as plsc

N = 10000
E = 320000
D = 128
NC = 2
NS = 16
NW = NC * NS
K = 80
EW = E // NW
NCHUNK = EW // K
NPAIR = (NCHUNK - 1) // 2
RPT = N // NS

_MESH = plsc.VectorSubcoreMesh(
    core_axis_name="c", subcore_axis_name="s", num_cores=NC, num_subcores=NS
)


def _make_sc_agg():
  scratch_types = [
      pltpu.VMEM((NCHUNK, K), jnp.int32),
      pltpu.VMEM((K,), jnp.int32),
      pltpu.VMEM((K,), jnp.int32),
      pltpu.VMEM((K, D), jnp.float32),
      pltpu.VMEM((K, D), jnp.float32),
      pltpu.SemaphoreType.DMA,
      pltpu.SemaphoreType.DMA,
      pltpu.SemaphoreType.DMA,
      pltpu.SemaphoreType.DMA,
      pltpu.VMEM_SHARED((N, D), jnp.float32),
  ]

  def body(x, srcr, dstr, zrows, agg_out, idx_s, idx_d0, idx_d1,
           rows0, rows1, sem0, sem1, dsem0, dsem1, agg_sh):
    c = lax.axis_index("c")
    s = lax.axis_index("s")
    wid = c * NS + s
    pltpu.sync_copy(zrows, agg_sh.at[pl.ds(s * RPT, RPT)])
    pltpu.sync_copy(srcr.at[wid], idx_s)
    plsc.subcore_barrier()

    last = NCHUNK - 1

    def gather(i, rows, sem):
      pltpu.async_copy(x.at[idx_s.at[i]], rows, sem)

    def dload(i, idx_d, dsem):
      pltpu.async_copy(dstr.at[wid, jnp.minimum(i, last)], idx_d, dsem)

    def scatter(rows, idx_d):
      pltpu.sync_copy(rows, agg_sh.at[idx_d], add=True)

    def gwait(rows, sem):
      pltpu.make_async_copy(x.at[idx_s.at[0]], rows, sem).wait()

    def dwait(idx_d, dsem):
      pltpu.make_async_copy(dstr.at[wid, 0], idx_d, dsem).wait()

    dload(0, idx_d0, dsem0)
    gather(0, rows0, sem0)
    dload(1, idx_d1, dsem1)

    def pair(j, carry):
      gather(2 * j + 1, rows1, sem1)
      gwait(rows0, sem0)
      dwait(idx_d0, dsem0)
      scatter(rows0, idx_d0)
      gather(2 * j + 2, rows0, sem0)
      dload(2 * j + 2, idx_d0, dsem0)
      gwait(rows1, sem1)
      dwait(idx_d1, dsem1)
      scatter(rows1, idx_d1)
      dload(2 * j + 3, idx_d1, dsem1)
      return carry

    lax.fori_loop(0, NPAIR, pair, 0)
    gwait(rows0, sem0)
    dwait(idx_d0, dsem0)
    scatter(rows0, idx_d0)
    dwait(idx_d1, dsem1)

    plsc.subcore_barrier()
    pltpu.sync_copy(agg_sh.at[pl.ds(s * RPT, RPT)], agg_out.at[c, s])

  return pl.kernel(
      body, mesh=_MESH, scratch_types=scratch_types, name="sage_sc_agg",
      out_type=[jax.ShapeDtypeStruct((NC, NS, RPT, D), jnp.float32)],
  )


def _make_sc_deg():
  scratch_types = [
      pltpu.VMEM((NCHUNK, K), jnp.int32),
      pltpu.VMEM((K, D), jnp.float32),
      pltpu.SemaphoreType.DMA,
      pltpu.VMEM_SHARED((N, D), jnp.float32),
  ]

  def body(dstr, zrows, onesr, deg_out, idx_d, ones_v, sem, deg_sh):
    c = lax.axis_index("c")
    s = lax.axis_index("s")
    wid = c * NS + s
    pltpu.sync_copy(zrows, deg_sh.at[pl.ds(s * RPT, RPT)])
    pltpu.sync_copy(onesr, ones_v)
    pltpu.sync_copy(dstr.at[wid], idx_d)
    plsc.subcore_barrier()

    def chunk(i, carry):
      pltpu.async_copy(ones_v, deg_sh.at[idx_d.at[i]], sem, add=True)
      return carry

    lax.fori_loop(0, NCHUNK, chunk, 0)

    def drain(i, carry):
      pltpu.make_async_copy(ones_v, deg_sh.at[idx_d.at[0]], sem).wait()
      return carry

    lax.fori_loop(0, NCHUNK, drain, 0)
    plsc.subcore_barrier()
    pltpu.sync_copy(deg_sh.at[pl.ds(s * RPT, RPT)], deg_out.at[c, s])

  return pl.kernel(
      body, mesh=_MESH, scratch_types=scratch_types, name="sage_sc_deg",
      out_type=[jax.ShapeDtypeStruct((NC, NS, RPT, D), jnp.float32)],
  )


def _make_dense(relu: bool):
  R = 1000

  def body(x_ref, a0_ref, a1_ref, d0_ref, d1_ref, ws_ref, wn_ref, b_ref,
           o_ref):
    deg = jnp.maximum(d0_ref[:, 0:1] + d1_ref[:, 0:1], 1.0)
    agg = (a0_ref[...] + a1_ref[...]) / deg
    dn = (((1,), (1,)), ((), ()))
    h = (
        lax.dot_general(x_ref[...], ws_ref[...], dn,
                        preferred_element_type=jnp.float32)
        + lax.dot_general(agg, wn_ref[...], dn,
                          preferred_element_type=jnp.float32)
        + b_ref[...]
    )
    o_ref[...] = jnp.maximum(h, 0.0) if relu else h

  row_block = lambda i: (i, 0)
  fixed = lambda i: (0, 0)
  return pl.pallas_call(
      body,
      grid=(N // R,),
      in_specs=[
          pl.BlockSpec((R, D), row_block),
          pl.BlockSpec((R, D), row_block),
          pl.BlockSpec((R, D), row_block),
          pl.BlockSpec((R, D), row_block),
          pl.BlockSpec((R, D), row_block),
          pl.BlockSpec((D, D), fixed),
          pl.BlockSpec((D, D), fixed),
          pl.BlockSpec((1, D), fixed),
      ],
      out_specs=pl.BlockSpec((R, D), row_block),
      out_shape=jax.ShapeDtypeStruct((N, D), jnp.float32),
  )


def kernel(x, edge_index, W1_self, W1_neigh, b1, W2_self, W2_neigh, b2):
  src = edge_index[0].astype(jnp.int32).reshape(NW, NCHUNK, K)
  dst = edge_index[1].astype(jnp.int32).reshape(NW, NCHUNK, K)
  zrows = jnp.zeros((RPT, D), jnp.float32)
  ones_rows = jnp.ones((K, D), jnp.float32)

  sc_agg = _make_sc_agg()

  (deg,) = _make_sc_deg()(dst, zrows, ones_rows)
  deg = deg.reshape(NC, N, D)

  (agg1,) = sc_agg(x, src, dst, zrows)
  agg1 = agg1.reshape(NC, N, D)
  h = _make_dense(True)(x, agg1[0], agg1[1], deg[0], deg[1],
                        W1_self, W1_neigh, b1.reshape(1, D))

  (agg2,) = sc_agg(h, src, dst, zrows)
  agg2 = agg2.reshape(NC, N, D)
  out = _make_dense(False)(h, agg2[0], agg2[1], deg[0], deg[1],
                           W2_self, W2_neigh, b2.reshape(1, D))
  return out


# speedup vs baseline: 9.3529x; 1.0283x over previous
import jax
import jax.numpy as jnp
from jax import lax
from jax.experimental import pallas as pl
from jax.experimental.pallas import tpu as pltpu
from jax.experimental.pallas import tpu_sc as plsc

N = 10000
E = 320000
D = 128
NC = 2
NS = 16
NW = NC * NS
K = 80
EW = E // NW
NCHUNK = EW // K
NPAIR = (NCHUNK - 1) // 2
RPT = N // NS

_MESH = plsc.VectorSubcoreMesh(
    core_axis_name="c", subcore_axis_name="s", num_cores=NC, num_subcores=NS
)


def _make_sc_agg():
  scratch_types = [
      pltpu.VMEM((NCHUNK, K), jnp.int32),
      pltpu.VMEM((K,), jnp.int32),
      pltpu.VMEM((K,), jnp.int32),
      pltpu.VMEM((K, D), jnp.float32),
      pltpu.VMEM((K, D), jnp.float32),
      pltpu.SemaphoreType.DMA,
      pltpu.SemaphoreType.DMA,
      pltpu.SemaphoreType.DMA,
      pltpu.SemaphoreType.DMA,
      pltpu.VMEM_SHARED((N, D), jnp.float32),
  ]

  def body(x, srcr, dstr, zrows, agg_out, idx_s, idx_d0, idx_d1,
           rows0, rows1, sem0, sem1, dsem0, dsem1, agg_sh):
    c = lax.axis_index("c")
    s = lax.axis_index("s")
    wid = c * NS + s
    pltpu.sync_copy(zrows, agg_sh.at[pl.ds(s * RPT, RPT)])
    pltpu.sync_copy(srcr.at[wid], idx_s)
    plsc.subcore_barrier()

    last = NCHUNK - 1

    def gather(i, rows, sem):
      pltpu.async_copy(x.at[idx_s.at[i]], rows, sem)

    def dload(i, idx_d, dsem):
      pltpu.async_copy(dstr.at[wid, jnp.minimum(i, last)], idx_d, dsem)

    def scatter(rows, idx_d):
      pltpu.sync_copy(rows, agg_sh.at[idx_d], add=True)

    def gwait(rows, sem):
      pltpu.make_async_copy(x.at[idx_s.at[0]], rows, sem).wait()

    def dwait(idx_d, dsem):
      pltpu.make_async_copy(dstr.at[wid, 0], idx_d, dsem).wait()

    dload(0, idx_d0, dsem0)
    gather(0, rows0, sem0)
    dload(1, idx_d1, dsem1)

    def pair(j, carry):
      gather(2 * j + 1, rows1, sem1)
      gwait(rows0, sem0)
      dwait(idx_d0, dsem0)
      scatter(rows0, idx_d0)
      gather(2 * j + 2, rows0, sem0)
      dload(2 * j + 2, idx_d0, dsem0)
      gwait(rows1, sem1)
      dwait(idx_d1, dsem1)
      scatter(rows1, idx_d1)
      dload(2 * j + 3, idx_d1, dsem1)
      return carry

    lax.fori_loop(0, NPAIR, pair, 0)
    gwait(rows0, sem0)
    dwait(idx_d0, dsem0)
    scatter(rows0, idx_d0)
    dwait(idx_d1, dsem1)

    plsc.subcore_barrier()
    pltpu.sync_copy(agg_sh.at[pl.ds(s * RPT, RPT)], agg_out.at[c, s])

  return pl.kernel(
      body, mesh=_MESH, scratch_types=scratch_types, name="sage_sc_agg",
      out_type=[jax.ShapeDtypeStruct((NC, NS, RPT, D), jnp.float32)],
  )


def _make_sc_deg():
  scratch_types = [
      pltpu.VMEM((NCHUNK, K), jnp.int32),
      pltpu.VMEM((K, D), jnp.float32),
      pltpu.SemaphoreType.DMA,
      pltpu.VMEM_SHARED((N, D), jnp.float32),
  ]

  def body(dstr, zrows, onesr, deg_out, idx_d, ones_v, sem, deg_sh):
    c = lax.axis_index("c")
    s = lax.axis_index("s")
    wid = c * NS + s
    pltpu.sync_copy(zrows, deg_sh.at[pl.ds(s * RPT, RPT)])
    pltpu.sync_copy(onesr, ones_v)
    pltpu.sync_copy(dstr.at[wid], idx_d)
    plsc.subcore_barrier()

    def chunk(i, carry):
      pltpu.async_copy(ones_v, deg_sh.at[idx_d.at[i]], sem, add=True)
      return carry

    lax.fori_loop(0, NCHUNK, chunk, 0)

    def drain(i, carry):
      pltpu.make_async_copy(ones_v, deg_sh.at[idx_d.at[0]], sem).wait()
      return carry

    lax.fori_loop(0, NCHUNK, drain, 0)
    plsc.subcore_barrier()
    pltpu.sync_copy(deg_sh.at[pl.ds(s * RPT, RPT)], deg_out.at[c, s])

  return pl.kernel(
      body, mesh=_MESH, scratch_types=scratch_types, name="sage_sc_deg",
      out_type=[jax.ShapeDtypeStruct((NC, NS, RPT, D), jnp.float32)],
  )


_R = 1000
_row_block = lambda i: (i, 0)
_fixed = lambda i: (0, 0)


def _make_self():
  def body(x_ref, ws_ref, b_ref, o_ref):
    dn = (((1,), (1,)), ((), ()))
    o_ref[...] = lax.dot_general(
        x_ref[...], ws_ref[...], dn, preferred_element_type=jnp.float32
    ) + b_ref[...]

  return pl.pallas_call(
      body,
      grid=(N // _R,),
      in_specs=[
          pl.BlockSpec((_R, D), _row_block),
          pl.BlockSpec((D, D), _fixed),
          pl.BlockSpec((1, D), _fixed),
      ],
      out_specs=pl.BlockSpec((_R, D), _row_block),
      out_shape=jax.ShapeDtypeStruct((N, D), jnp.float32),
  )


def _make_combine(relu: bool):
  def body(s_ref, a0_ref, a1_ref, d_ref, wn_ref, o_ref):
    deg = jnp.maximum(d_ref[:, 0:1], 1.0)
    agg = (a0_ref[...] + a1_ref[...]) / deg
    dn = (((1,), (1,)), ((), ()))
    h = s_ref[...] + lax.dot_general(
        agg, wn_ref[...], dn, preferred_element_type=jnp.float32
    )
    o_ref[...] = jnp.maximum(h, 0.0) if relu else h

  return pl.pallas_call(
      body,
      grid=(N // _R,),
      in_specs=[
          pl.BlockSpec((_R, D), _row_block),
          pl.BlockSpec((_R, D), _row_block),
          pl.BlockSpec((_R, D), _row_block),
          pl.BlockSpec((_R, 1), _row_block),
          pl.BlockSpec((D, D), _fixed),
      ],
      out_specs=pl.BlockSpec((_R, D), _row_block),
      out_shape=jax.ShapeDtypeStruct((N, D), jnp.float32),
  )


def kernel(x, edge_index, W1_self, W1_neigh, b1, W2_self, W2_neigh, b2):
  src = edge_index[0].astype(jnp.int32).reshape(NW, NCHUNK, K)
  dst = edge_index[1].astype(jnp.int32).reshape(NW, NCHUNK, K)
  zrows = jnp.zeros((RPT, D), jnp.float32)
  ones_rows = jnp.ones((K, D), jnp.float32)

  sc_agg = _make_sc_agg()
  dense_self = _make_self()

  (deg,) = _make_sc_deg()(dst, zrows, ones_rows)
  deg = deg.reshape(NC, N, D)
  degc = deg[0, :, 0:1] + deg[1, :, 0:1]

  (agg1,) = sc_agg(x, src, dst, zrows)
  agg1 = agg1.reshape(NC, N, D)
  s1 = dense_self(x, W1_self, b1.reshape(1, D))
  h = _make_combine(True)(s1, agg1[0], agg1[1], degc, W1_neigh)

  (agg2,) = sc_agg(h, src, dst, zrows)
  agg2 = agg2.reshape(NC, N, D)
  s2 = dense_self(h, W2_self, b2.reshape(1, D))
  out = _make_combine(False)(s2, agg2[0], agg2[1], degc, W2_neigh)
  return out
